# Initial kernel scaffold; baseline (speedup 1.0000x reference)
import jax
import jax.numpy as jnp
from jax.experimental import pallas as pl


def kernel(x, edge_index, params):
    raise NotImplementedError("write your pallas kernel here")


# speedup vs baseline: 9.1034x; 9.1034x over previous
import functools

import jax
import jax.numpy as jnp
from jax import lax
from jax.experimental import pallas as pl
from jax.experimental.pallas import tpu as pltpu
import jax.experimental.pallas.tpu_sc as plsc

_N = 10000
_E = 320000
_IN = 128
_H = 64
_L = 4
_NP = 10240
_EP = 327680
_NC = 2
_NS = 16
_NW = _NC * _NS
_EPW = _EP // _NW
_BB = 128
_NB = _EPW // _BB
_RPT = _NP // _NS
_DW = 16


def _degree_body(dst_hbm, ones_hbm, z_hbm, out_hbm, dst_v, ones_v, deg_sh):
    c = lax.axis_index("c")
    s = lax.axis_index("s")
    wid = s * _NC + c
    pltpu.sync_copy(z_hbm.at[pl.ds(s * _RPT, _RPT)],
                    deg_sh.at[pl.ds(s * _RPT, _RPT)])
    pltpu.sync_copy(dst_hbm.at[wid], dst_v)
    pltpu.sync_copy(ones_hbm, ones_v)
    plsc.subcore_barrier()

    def body(j, carry):
        pltpu.sync_copy(ones_v, deg_sh.at[dst_v.at[j]], add=True)
        return carry

    lax.fori_loop(0, _NB, body, 0)
    plsc.subcore_barrier()
    pltpu.sync_copy(deg_sh.at[pl.ds(s * _RPT, _RPT)],
                    out_hbm.at[c, pl.ds(s * _RPT, _RPT)])


def _sc_degree(dstp, ones, zeros):
    mesh = plsc.VectorSubcoreMesh(core_axis_name="c", subcore_axis_name="s")
    f = pl.kernel(
        _degree_body,
        out_type=jax.ShapeDtypeStruct((_NC, _NP, _DW), jnp.float32),
        mesh=mesh,
        scratch_types=[
            pltpu.VMEM((_NB, _BB), jnp.int32),
            pltpu.VMEM((_BB, _DW), jnp.float32),
            pltpu.VMEM_SHARED((_NP, _DW), jnp.float32),
        ],
        compiler_params=pltpu.CompilerParams(use_tc_tiling_on_sc=False),
    )
    return f(dstp, ones, zeros)


def _scatter_body(y_hbm, src_hbm, dst_hbm, z_hbm, out_hbm,
                  src_v, dst_v, rows_v, acc_sh, sem):
    c = lax.axis_index("c")
    s = lax.axis_index("s")
    wid = s * _NC + c
    pltpu.sync_copy(z_hbm.at[pl.ds(s * _RPT, _RPT)],
                    acc_sh.at[pl.ds(s * _RPT, _RPT)])
    pltpu.sync_copy(src_hbm.at[wid], src_v)
    pltpu.sync_copy(dst_hbm.at[wid], dst_v)
    plsc.subcore_barrier()

    def body(j, carry):
        pltpu.async_copy(y_hbm.at[src_v.at[j]], rows_v, sem).wait()
        pltpu.sync_copy(rows_v, acc_sh.at[dst_v.at[j]], add=True)
        return carry

    lax.fori_loop(0, _NB, body, 0)
    plsc.subcore_barrier()
    pltpu.sync_copy(acc_sh.at[pl.ds(s * _RPT, _RPT)],
                    out_hbm.at[c, pl.ds(s * _RPT, _RPT)])


def _sc_scatter(y, srcp, dstp, zeros):
    mesh = plsc.VectorSubcoreMesh(core_axis_name="c", subcore_axis_name="s")
    f = pl.kernel(
        _scatter_body,
        out_type=jax.ShapeDtypeStruct((_NC, _NP, _H), jnp.float32),
        mesh=mesh,
        scratch_types=[
            pltpu.VMEM((_NB, _BB), jnp.int32),
            pltpu.VMEM((_NB, _BB), jnp.int32),
            pltpu.VMEM((_BB, _H), jnp.float32),
            pltpu.VMEM_SHARED((_NP, _H), jnp.float32),
            pltpu.SemaphoreType.DMA,
        ],
        compiler_params=pltpu.CompilerParams(use_tc_tiling_on_sc=False),
    )
    return f(y, srcp, dstp, zeros)


def _dot(a, b):
    return jnp.dot(a, b, preferred_element_type=jnp.float32)


def _fe_body(x_ref, deg_ref, w1_ref, b1_ref, w2_ref, b2_ref, w0_ref,
             dinv_ref, y0_ref):
    deg = deg_ref[0, :, 0:1] + deg_ref[1, :, 0:1] + 1.0
    dinv = lax.rsqrt(deg)
    h = jnp.maximum(_dot(x_ref[...], w1_ref[...]) + b1_ref[...], 0.0)
    h = _dot(h, w2_ref[...]) + b2_ref[...]
    dinv_ref[...] = dinv
    y0_ref[...] = dinv * _dot(h, w0_ref[...])


def _tc_fe(xp, degp, p):
    return pl.pallas_call(
        _fe_body,
        out_shape=[
            jax.ShapeDtypeStruct((_NP, 1), jnp.float32),
            jax.ShapeDtypeStruct((_NP, _H), jnp.float32),
        ],
    )(xp, degp, p['fe_W1'], p['fe_b1'].reshape(1, -1), p['fe_W2'],
      p['fe_b2'].reshape(1, -1), p['gcn_W0'])


def _bn_block(accp_ref, y_ref, dinv_ref, gb_ref, bng_ref, bnb_ref):
    dinv = dinv_ref[...]
    t = dinv * (accp_ref[0] + accp_ref[1] + y_ref[...]) + gb_ref[...]
    mask = (lax.broadcasted_iota(jnp.int32, (_NP, 1), 0) < _N
            ).astype(jnp.float32)
    m = jnp.sum(t * mask, axis=0, keepdims=True) * (1.0 / _N)
    d = (t - m) * mask
    v = jnp.sum(d * d, axis=0, keepdims=True) * (1.0 / _N)
    return bng_ref[...] * (t - m) * lax.rsqrt(v + 1e-5) + bnb_ref[...], dinv


def _mid_body(accp_ref, y_ref, dinv_ref, gb_ref, bng_ref, bnb_ref, wn_ref,
              ynext_ref):
    xh, dinv = _bn_block(accp_ref, y_ref, dinv_ref, gb_ref, bng_ref, bnb_ref)
    ynext_ref[...] = dinv * _dot(jnp.maximum(xh, 0.0), wn_ref[...])


def _tc_mid(accp, y, dinv, p, i):
    return pl.pallas_call(
        _mid_body,
        out_shape=jax.ShapeDtypeStruct((_NP, _H), jnp.float32),
    )(accp, y, dinv, p[f'gcn_b{i}'].reshape(1, -1),
      p[f'bn_g{i}'].reshape(1, -1), p[f'bn_b{i}'].reshape(1, -1),
      p[f'gcn_W{i + 1}'])


def _final_body(accp_ref, y_ref, dinv_ref, gb_ref, bng_ref, bnb_ref,
                cw1_ref, cb1_ref, cw2_ref, cb2_ref,
                fw1_ref, fb1_ref, fw2_ref, fb2_ref,
                rw1_ref, rb1_ref, rw2_ref, rb2_ref,
                logits_ref, conf_ref, corr_ref):
    h, _ = _bn_block(accp_ref, y_ref, dinv_ref, gb_ref, bng_ref, bnb_ref)
    cc = jnp.maximum(_dot(h, cw1_ref[...]) + cb1_ref[...], 0.0)
    logits_ref[...] = _dot(cc, cw2_ref[...]) + cb2_ref[...]
    cf = jnp.maximum(_dot(h, fw1_ref[...]) + fb1_ref[...], 0.0)
    conf_ref[...] = jax.nn.sigmoid(_dot(cf, fw2_ref[...]) + fb2_ref[...])
    cr = jnp.maximum(_dot(h, rw1_ref[...]) + rb1_ref[...], 0.0)
    corr_ref[...] = _dot(cr, rw2_ref[...]) + rb2_ref[...]


def _tc_final(accp, y, dinv, p):
    i = _L - 1
    return pl.pallas_call(
        _final_body,
        out_shape=[
            jax.ShapeDtypeStruct((_NP, 3), jnp.float32),
            jax.ShapeDtypeStruct((_NP, 1), jnp.float32),
            jax.ShapeDtypeStruct((_NP, 1), jnp.float32),
        ],
    )(accp, y, dinv, p[f'gcn_b{i}'].reshape(1, -1),
      p[f'bn_g{i}'].reshape(1, -1), p[f'bn_b{i}'].reshape(1, -1),
      p['cls_W1'], p['cls_b1'].reshape(1, -1),
      p['cls_W2'], p['cls_b2'].reshape(1, -1),
      p['conf_W1'], p['conf_b1'].reshape(1, -1),
      p['conf_W2'], p['conf_b2'].reshape(1, -1),
      p['corr_W1'], p['corr_b1'].reshape(1, -1),
      p['corr_W2'], p['corr_b2'].reshape(1, -1))


def kernel(x, edge_index, params):
    p = params
    pad = jnp.full((_EP - _E,), _NP - 1, jnp.int32)
    srcp = jnp.concatenate([edge_index[0], pad]).reshape(_NW, _NB, _BB)
    dstp = jnp.concatenate([edge_index[1], pad]).reshape(_NW, _NB, _BB)
    xp = jnp.pad(x, ((0, _NP - _N), (0, 0)))
    zeros_deg = jnp.zeros((_NP, _DW), jnp.float32)
    ones_deg = jnp.ones((_BB, _DW), jnp.float32)
    zeros_acc = jnp.zeros((_NP, _H), jnp.float32)

    degp = _sc_degree(dstp, ones_deg, zeros_deg)
    dinv, y = _tc_fe(xp, degp, p)
    for i in range(_L - 1):
        accp = _sc_scatter(y, srcp, dstp, zeros_acc)
        y = _tc_mid(accp, y, dinv, p, i)
    accp = _sc_scatter(y, srcp, dstp, zeros_acc)
    logits, conf, corr = _tc_final(accp, y, dinv, p)
    return (logits[:_N], conf[:_N, 0], corr[:_N, 0])


# speedup vs baseline: 9.9889x; 1.0973x over previous
import functools

import jax
import jax.numpy as jnp
from jax import lax
from jax.experimental import pallas as pl
from jax.experimental.pallas import tpu as pltpu
import jax.experimental.pallas.tpu_sc as plsc

_N = 10000
_E = 320000
_IN = 128
_H = 64
_L = 4
_NP = 10240
_EP = 327680
_NC = 2
_NS = 16
_NW = _NC * _NS
_EPW = _EP // _NW
_BB = 128
_NB = _EPW // _BB
_RPT = _NP // _NS
_DW = 16


def _degree_body(dst_hbm, ones_hbm, z_hbm, out_hbm, dst_v, ones_v, deg_sh):
    c = lax.axis_index("c")
    s = lax.axis_index("s")
    wid = s * _NC + c
    pltpu.sync_copy(z_hbm.at[pl.ds(s * _RPT, _RPT)],
                    deg_sh.at[pl.ds(s * _RPT, _RPT)])
    pltpu.sync_copy(dst_hbm.at[wid], dst_v)
    pltpu.sync_copy(ones_hbm, ones_v)
    plsc.subcore_barrier()

    def body(j, carry):
        pltpu.sync_copy(ones_v, deg_sh.at[dst_v.at[j]], add=True)
        return carry

    lax.fori_loop(0, _NB, body, 0)
    plsc.subcore_barrier()
    pltpu.sync_copy(deg_sh.at[pl.ds(s * _RPT, _RPT)],
                    out_hbm.at[c, pl.ds(s * _RPT, _RPT)])


def _sc_degree(dstp, ones, zeros):
    mesh = plsc.VectorSubcoreMesh(core_axis_name="c", subcore_axis_name="s")
    f = pl.kernel(
        _degree_body,
        out_type=jax.ShapeDtypeStruct((_NC, _NP, _DW), jnp.float32),
        mesh=mesh,
        scratch_types=[
            pltpu.VMEM((_NB, _BB), jnp.int32),
            pltpu.VMEM((_BB, _DW), jnp.float32),
            pltpu.VMEM_SHARED((_NP, _DW), jnp.float32),
        ],
        compiler_params=pltpu.CompilerParams(use_tc_tiling_on_sc=False),
    )
    return f(dstp, ones, zeros)


_KB = 4
_NBUF = 2
_NSUP = _NB // (_KB * _NBUF)


def _scatter_body(y_hbm, src_hbm, dst_hbm, z_hbm, out_hbm,
                  src_v, dst_v, rows0, rows1, acc_sh, sg0, sg1, ss0, ss1):
    c = lax.axis_index("c")
    s = lax.axis_index("s")
    wid = s * _NC + c
    pltpu.sync_copy(z_hbm.at[pl.ds(s * _RPT, _RPT)],
                    acc_sh.at[pl.ds(s * _RPT, _RPT)])
    pltpu.sync_copy(src_hbm.at[wid], src_v)
    pltpu.sync_copy(dst_hbm.at[wid], dst_v)
    plsc.subcore_barrier()

    def sup(i, carry):
        b0 = i * (_KB * _NBUF)
        g0 = [pltpu.async_copy(y_hbm.at[src_v.at[b0 + k]],
                               rows0.at[pl.ds(k * _BB, _BB)], sg0)
              for k in range(_KB)]
        g1 = [pltpu.async_copy(y_hbm.at[src_v.at[b0 + _KB + k]],
                               rows1.at[pl.ds(k * _BB, _BB)], sg1)
              for k in range(_KB)]
        for d in g0:
            d.wait()
        s0 = [pltpu.async_copy(rows0.at[pl.ds(k * _BB, _BB)],
                               acc_sh.at[dst_v.at[b0 + k]], ss0, add=True)
              for k in range(_KB)]
        for d in g1:
            d.wait()
        s1 = [pltpu.async_copy(rows1.at[pl.ds(k * _BB, _BB)],
                               acc_sh.at[dst_v.at[b0 + _KB + k]], ss1,
                               add=True)
              for k in range(_KB)]
        for d in s0:
            d.wait()
        for d in s1:
            d.wait()
        return carry

    lax.fori_loop(0, _NSUP, sup, 0)
    plsc.subcore_barrier()
    pltpu.sync_copy(acc_sh.at[pl.ds(s * _RPT, _RPT)],
                    out_hbm.at[c, pl.ds(s * _RPT, _RPT)])


def _sc_scatter(y, srcp, dstp, zeros):
    mesh = plsc.VectorSubcoreMesh(core_axis_name="c", subcore_axis_name="s")
    f = pl.kernel(
        _scatter_body,
        out_type=jax.ShapeDtypeStruct((_NC, _NP, _H), jnp.float32),
        mesh=mesh,
        scratch_types=[
            pltpu.VMEM((_NB, _BB), jnp.int32),
            pltpu.VMEM((_NB, _BB), jnp.int32),
            pltpu.VMEM((_KB * _BB, _H), jnp.float32),
            pltpu.VMEM((_KB * _BB, _H), jnp.float32),
            pltpu.VMEM_SHARED((_NP, _H), jnp.float32),
            pltpu.SemaphoreType.DMA,
            pltpu.SemaphoreType.DMA,
            pltpu.SemaphoreType.DMA,
            pltpu.SemaphoreType.DMA,
        ],
        compiler_params=pltpu.CompilerParams(use_tc_tiling_on_sc=False),
    )
    return f(y, srcp, dstp, zeros)


def _dot(a, b):
    return jnp.dot(a, b, preferred_element_type=jnp.float32)


def _fe_body(x_ref, deg_ref, w1_ref, b1_ref, w2_ref, b2_ref, w0_ref,
             dinv_ref, y0_ref):
    deg = deg_ref[0, :, 0:1] + deg_ref[1, :, 0:1] + 1.0
    dinv = lax.rsqrt(deg)
    h = jnp.maximum(_dot(x_ref[...], w1_ref[...]) + b1_ref[...], 0.0)
    h = _dot(h, w2_ref[...]) + b2_ref[...]
    dinv_ref[...] = dinv
    y0_ref[...] = dinv * _dot(h, w0_ref[...])


def _tc_fe(xp, degp, p):
    return pl.pallas_call(
        _fe_body,
        out_shape=[
            jax.ShapeDtypeStruct((_NP, 1), jnp.float32),
            jax.ShapeDtypeStruct((_NP, _H), jnp.float32),
        ],
    )(xp, degp, p['fe_W1'], p['fe_b1'].reshape(1, -1), p['fe_W2'],
      p['fe_b2'].reshape(1, -1), p['gcn_W0'])


def _bn_block(accp_ref, y_ref, dinv_ref, gb_ref, bng_ref, bnb_ref):
    dinv = dinv_ref[...]
    t = dinv * (accp_ref[0] + accp_ref[1] + y_ref[...]) + gb_ref[...]
    mask = (lax.broadcasted_iota(jnp.int32, (_NP, 1), 0) < _N
            ).astype(jnp.float32)
    m = jnp.sum(t * mask, axis=0, keepdims=True) * (1.0 / _N)
    d = (t - m) * mask
    v = jnp.sum(d * d, axis=0, keepdims=True) * (1.0 / _N)
    return bng_ref[...] * (t - m) * lax.rsqrt(v + 1e-5) + bnb_ref[...], dinv


def _mid_body(accp_ref, y_ref, dinv_ref, gb_ref, bng_ref, bnb_ref, wn_ref,
              ynext_ref):
    xh, dinv = _bn_block(accp_ref, y_ref, dinv_ref, gb_ref, bng_ref, bnb_ref)
    ynext_ref[...] = dinv * _dot(jnp.maximum(xh, 0.0), wn_ref[...])


def _tc_mid(accp, y, dinv, p, i):
    return pl.pallas_call(
        _mid_body,
        out_shape=jax.ShapeDtypeStruct((_NP, _H), jnp.float32),
    )(accp, y, dinv, p[f'gcn_b{i}'].reshape(1, -1),
      p[f'bn_g{i}'].reshape(1, -1), p[f'bn_b{i}'].reshape(1, -1),
      p[f'gcn_W{i + 1}'])


def _final_body(accp_ref, y_ref, dinv_ref, gb_ref, bng_ref, bnb_ref,
                cw1_ref, cb1_ref, cw2_ref, cb2_ref,
                fw1_ref, fb1_ref, fw2_ref, fb2_ref,
                rw1_ref, rb1_ref, rw2_ref, rb2_ref,
                logits_ref, conf_ref, corr_ref):
    h, _ = _bn_block(accp_ref, y_ref, dinv_ref, gb_ref, bng_ref, bnb_ref)
    cc = jnp.maximum(_dot(h, cw1_ref[...]) + cb1_ref[...], 0.0)
    logits_ref[...] = _dot(cc, cw2_ref[...]) + cb2_ref[...]
    cf = jnp.maximum(_dot(h, fw1_ref[...]) + fb1_ref[...], 0.0)
    conf_ref[...] = jax.nn.sigmoid(_dot(cf, fw2_ref[...]) + fb2_ref[...])
    cr = jnp.maximum(_dot(h, rw1_ref[...]) + rb1_ref[...], 0.0)
    corr_ref[...] = _dot(cr, rw2_ref[...]) + rb2_ref[...]


def _tc_final(accp, y, dinv, p):
    i = _L - 1
    return pl.pallas_call(
        _final_body,
        out_shape=[
            jax.ShapeDtypeStruct((_NP, 3), jnp.float32),
            jax.ShapeDtypeStruct((_NP, 1), jnp.float32),
            jax.ShapeDtypeStruct((_NP, 1), jnp.float32),
        ],
    )(accp, y, dinv, p[f'gcn_b{i}'].reshape(1, -1),
      p[f'bn_g{i}'].reshape(1, -1), p[f'bn_b{i}'].reshape(1, -1),
      p['cls_W1'], p['cls_b1'].reshape(1, -1),
      p['cls_W2'], p['cls_b2'].reshape(1, -1),
      p['conf_W1'], p['conf_b1'].reshape(1, -1),
      p['conf_W2'], p['conf_b2'].reshape(1, -1),
      p['corr_W1'], p['corr_b1'].reshape(1, -1),
      p['corr_W2'], p['corr_b2'].reshape(1, -1))


def kernel(x, edge_index, params):
    p = params
    pad = jnp.full((_EP - _E,), _NP - 1, jnp.int32)
    srcp = jnp.concatenate([edge_index[0], pad]).reshape(_NW, _NB, _BB)
    dstp = jnp.concatenate([edge_index[1], pad]).reshape(_NW, _NB, _BB)
    xp = jnp.pad(x, ((0, _NP - _N), (0, 0)))
    zeros_deg = jnp.zeros((_NP, _DW), jnp.float32)
    ones_deg = jnp.ones((_BB, _DW), jnp.float32)
    zeros_acc = jnp.zeros((_NP, _H), jnp.float32)

    degp = _sc_degree(dstp, ones_deg, zeros_deg)
    dinv, y = _tc_fe(xp, degp, p)
    for i in range(_L - 1):
        accp = _sc_scatter(y, srcp, dstp, zeros_acc)
        y = _tc_mid(accp, y, dinv, p, i)
    accp = _sc_scatter(y, srcp, dstp, zeros_acc)
    logits, conf, corr = _tc_final(accp, y, dinv, p)
    return (logits[:_N], conf[:_N, 0], corr[:_N, 0])


# speedup vs baseline: 28.3841x; 2.8416x over previous
import functools

import jax
import jax.numpy as jnp
from jax import lax
from jax.experimental import pallas as pl
from jax.experimental.pallas import tpu as pltpu
import jax.experimental.pallas.tpu_sc as plsc

_N = 10000
_E = 320000
_IN = 128
_H = 64
_L = 4
_NP = 10240
_EP = 327680
_NC = 2
_NS = 16
_NW = _NC * _NS
_EPW = _EP // _NW
_BB = 128
_NB = _EPW // _BB
_RPT = _NP // _NS
_DW = 16


def _degree_body(dst_hbm, ones_hbm, z_hbm, out_hbm, dst_v, ones_v, deg_sh):
    c = lax.axis_index("c")
    s = lax.axis_index("s")
    wid = s * _NC + c
    pltpu.sync_copy(z_hbm.at[pl.ds(s * _RPT, _RPT)],
                    deg_sh.at[pl.ds(s * _RPT, _RPT)])
    pltpu.sync_copy(dst_hbm.at[wid], dst_v)
    pltpu.sync_copy(ones_hbm, ones_v)
    plsc.subcore_barrier()

    def body(j, carry):
        pltpu.sync_copy(ones_v, deg_sh.at[dst_v.at[j]], add=True)
        return carry

    lax.fori_loop(0, _NB, body, 0)
    plsc.subcore_barrier()
    pltpu.sync_copy(deg_sh.at[pl.ds(s * _RPT, _RPT)],
                    out_hbm.at[c, pl.ds(s * _RPT, _RPT)])


def _sc_degree(dstp, ones, zeros):
    mesh = plsc.VectorSubcoreMesh(core_axis_name="c", subcore_axis_name="s")
    f = pl.kernel(
        _degree_body,
        out_type=jax.ShapeDtypeStruct((_NC, _NP, _DW), jnp.float32),
        mesh=mesh,
        scratch_types=[
            pltpu.VMEM((_NB, _BB), jnp.int32),
            pltpu.VMEM((_BB, _DW), jnp.float32),
            pltpu.VMEM_SHARED((_NP, _DW), jnp.float32),
        ],
        compiler_params=pltpu.CompilerParams(use_tc_tiling_on_sc=False),
    )
    return f(dstp, ones, zeros)


_KB = 4
_NBUF = 2
_NSUP = _NB // (_KB * _NBUF)


def _scatter_body(y_hbm, src_hbm, dst_hbm, z_hbm, out_hbm,
                  src_v, dst_v, rows0, rows1, acc_sh, sg0, sg1, ss0, ss1):
    c = lax.axis_index("c")
    s = lax.axis_index("s")
    wid = s * _NC + c
    pltpu.sync_copy(z_hbm.at[pl.ds(s * _RPT, _RPT)],
                    acc_sh.at[pl.ds(s * _RPT, _RPT)])
    pltpu.sync_copy(src_hbm.at[wid], src_v)
    pltpu.sync_copy(dst_hbm.at[wid], dst_v)
    plsc.subcore_barrier()

    def sup(i, carry):
        b0 = i * (_KB * _NBUF)
        g0 = [pltpu.async_copy(y_hbm.at[src_v.at[b0 + k]],
                               rows0.at[pl.ds(k * _BB, _BB)], sg0)
              for k in range(_KB)]
        g1 = [pltpu.async_copy(y_hbm.at[src_v.at[b0 + _KB + k]],
                               rows1.at[pl.ds(k * _BB, _BB)], sg1)
              for k in range(_KB)]
        for d in g0:
            d.wait()
        s0 = [pltpu.async_copy(rows0.at[pl.ds(k * _BB, _BB)],
                               acc_sh.at[dst_v.at[b0 + k]], ss0, add=True)
              for k in range(_KB)]
        for d in g1:
            d.wait()
        s1 = [pltpu.async_copy(rows1.at[pl.ds(k * _BB, _BB)],
                               acc_sh.at[dst_v.at[b0 + _KB + k]], ss1,
                               add=True)
              for k in range(_KB)]
        for d in s0:
            d.wait()
        for d in s1:
            d.wait()
        return carry

    lax.fori_loop(0, _NSUP, sup, 0)
    plsc.subcore_barrier()
    pltpu.sync_copy(acc_sh.at[pl.ds(s * _RPT, _RPT)],
                    out_hbm.at[c, pl.ds(s * _RPT, _RPT)])


def _sc_scatter(y, srcp, dstp, zeros):
    mesh = plsc.VectorSubcoreMesh(core_axis_name="c", subcore_axis_name="s")
    f = pl.kernel(
        _scatter_body,
        out_type=jax.ShapeDtypeStruct((_NC, _NP, _H), jnp.float32),
        mesh=mesh,
        scratch_types=[
            pltpu.VMEM((_NB, _BB), jnp.int32),
            pltpu.VMEM((_NB, _BB), jnp.int32),
            pltpu.VMEM((_KB * _BB, _H), jnp.float32),
            pltpu.VMEM((_KB * _BB, _H), jnp.float32),
            pltpu.VMEM_SHARED((_NP, _H), jnp.float32),
            pltpu.SemaphoreType.DMA,
            pltpu.SemaphoreType.DMA,
            pltpu.SemaphoreType.DMA,
            pltpu.SemaphoreType.DMA,
        ],
        compiler_params=pltpu.CompilerParams(use_tc_tiling_on_sc=False),
    )
    return f(y, srcp, dstp, zeros)


def _dot(a, b):
    return jnp.dot(a, b, preferred_element_type=jnp.float32)


def _fe_body(x_ref, deg_ref, w1_ref, b1_ref, w2_ref, b2_ref, w0_ref,
             dinv_ref, y0_ref):
    deg = deg_ref[0, :, 0:1] + deg_ref[1, :, 0:1] + 1.0
    dinv = lax.rsqrt(deg)
    h = jnp.maximum(_dot(x_ref[...], w1_ref[...]) + b1_ref[...], 0.0)
    h = _dot(h, w2_ref[...]) + b2_ref[...]
    dinv_ref[...] = dinv
    y0_ref[...] = dinv * _dot(h, w0_ref[...])


def _tc_fe(xp, degp, p):
    return pl.pallas_call(
        _fe_body,
        out_shape=[
            jax.ShapeDtypeStruct((_NP, 1), jnp.float32),
            jax.ShapeDtypeStruct((_NP, _H), jnp.float32),
        ],
    )(xp, degp, p['fe_W1'], p['fe_b1'].reshape(1, -1), p['fe_W2'],
      p['fe_b2'].reshape(1, -1), p['gcn_W0'])


def _bn_block(accp_ref, y_ref, dinv_ref, gb_ref, bng_ref, bnb_ref):
    dinv = dinv_ref[...]
    t = dinv * (accp_ref[0] + accp_ref[1] + y_ref[...]) + gb_ref[...]
    mask = (lax.broadcasted_iota(jnp.int32, (_NP, 1), 0) < _N
            ).astype(jnp.float32)
    m = jnp.sum(t * mask, axis=0, keepdims=True) * (1.0 / _N)
    d = (t - m) * mask
    v = jnp.sum(d * d, axis=0, keepdims=True) * (1.0 / _N)
    return bng_ref[...] * (t - m) * lax.rsqrt(v + 1e-5) + bnb_ref[...], dinv


def _mid_body(accp_ref, y_ref, dinv_ref, gb_ref, bng_ref, bnb_ref, wn_ref,
              ynext_ref):
    xh, dinv = _bn_block(accp_ref, y_ref, dinv_ref, gb_ref, bng_ref, bnb_ref)
    ynext_ref[...] = dinv * _dot(jnp.maximum(xh, 0.0), wn_ref[...])


def _tc_mid(accp, y, dinv, p, i):
    return pl.pallas_call(
        _mid_body,
        out_shape=jax.ShapeDtypeStruct((_NP, _H), jnp.float32),
    )(accp, y, dinv, p[f'gcn_b{i}'].reshape(1, -1),
      p[f'bn_g{i}'].reshape(1, -1), p[f'bn_b{i}'].reshape(1, -1),
      p[f'gcn_W{i + 1}'])


def _final_body(accp_ref, y_ref, dinv_ref, gb_ref, bng_ref, bnb_ref,
                cw1_ref, cb1_ref, cw2_ref, cb2_ref,
                fw1_ref, fb1_ref, fw2_ref, fb2_ref,
                rw1_ref, rb1_ref, rw2_ref, rb2_ref,
                logits_ref, conf_ref, corr_ref):
    h, _ = _bn_block(accp_ref, y_ref, dinv_ref, gb_ref, bng_ref, bnb_ref)
    cc = jnp.maximum(_dot(h, cw1_ref[...]) + cb1_ref[...], 0.0)
    logits_ref[...] = _dot(cc, cw2_ref[...]) + cb2_ref[...]
    cf = jnp.maximum(_dot(h, fw1_ref[...]) + fb1_ref[...], 0.0)
    conf_ref[...] = jax.nn.sigmoid(_dot(cf, fw2_ref[...]) + fb2_ref[...])
    cr = jnp.maximum(_dot(h, rw1_ref[...]) + rb1_ref[...], 0.0)
    corr_ref[...] = _dot(cr, rw2_ref[...]) + rb2_ref[...]


def _tc_final(accp, y, dinv, p):
    i = _L - 1
    return pl.pallas_call(
        _final_body,
        out_shape=[
            jax.ShapeDtypeStruct((_NP, 3), jnp.float32),
            jax.ShapeDtypeStruct((_NP, 1), jnp.float32),
            jax.ShapeDtypeStruct((_NP, 1), jnp.float32),
        ],
    )(accp, y, dinv, p[f'gcn_b{i}'].reshape(1, -1),
      p[f'bn_g{i}'].reshape(1, -1), p[f'bn_b{i}'].reshape(1, -1),
      p['cls_W1'], p['cls_b1'].reshape(1, -1),
      p['cls_W2'], p['cls_b2'].reshape(1, -1),
      p['conf_W1'], p['conf_b1'].reshape(1, -1),
      p['conf_W2'], p['conf_b2'].reshape(1, -1),
      p['corr_W1'], p['corr_b1'].reshape(1, -1),
      p['corr_W2'], p['corr_b2'].reshape(1, -1))


def kernel(x, edge_index, params):
    p = params
    pad = _N + (jnp.arange(_EP - _E, dtype=jnp.int32) % (_NP - _N))
    srcp = jnp.concatenate([edge_index[0], pad]).reshape(_NW, _NB, _BB)
    dstp = jnp.concatenate([edge_index[1], pad]).reshape(_NW, _NB, _BB)
    xp = jnp.pad(x, ((0, _NP - _N), (0, 0)))
    zeros_deg = jnp.zeros((_NP, _DW), jnp.float32)
    ones_deg = jnp.ones((_BB, _DW), jnp.float32)
    zeros_acc = jnp.zeros((_NP, _H), jnp.float32)

    degp = _sc_degree(dstp, ones_deg, zeros_deg)
    dinv, y = _tc_fe(xp, degp, p)
    for i in range(_L - 1):
        accp = _sc_scatter(y, srcp, dstp, zeros_acc)
        y = _tc_mid(accp, y, dinv, p, i)
    accp = _sc_scatter(y, srcp, dstp, zeros_acc)
    logits, conf, corr = _tc_final(accp, y, dinv, p)
    return (logits[:_N], conf[:_N, 0], corr[:_N, 0])


# speedup vs baseline: 28.5712x; 1.0066x over previous
import functools

import jax
import jax.numpy as jnp
from jax import lax
from jax.experimental import pallas as pl
from jax.experimental.pallas import tpu as pltpu
import jax.experimental.pallas.tpu_sc as plsc

_N = 10000
_E = 320000
_IN = 128
_H = 64
_L = 4
_NP = 10240
_EP = 327680
_NC = 2
_NS = 16
_NW = _NC * _NS
_EPW = _EP // _NW
_BB = 128
_NB = _EPW // _BB
_RPT = _NP // _NS
_DW = 16


def _degree_body(dst_hbm, ones_hbm, z_hbm, out_hbm, dst_v, ones_v, deg_sh):
    c = lax.axis_index("c")
    s = lax.axis_index("s")
    wid = s * _NC + c
    pltpu.sync_copy(z_hbm.at[pl.ds(s * _RPT, _RPT)],
                    deg_sh.at[pl.ds(s * _RPT, _RPT)])
    pltpu.sync_copy(dst_hbm.at[wid], dst_v)
    pltpu.sync_copy(ones_hbm, ones_v)
    plsc.subcore_barrier()

    def body(j, carry):
        pltpu.sync_copy(ones_v, deg_sh.at[dst_v.at[j]], add=True)
        return carry

    lax.fori_loop(0, _NB, body, 0)
    plsc.subcore_barrier()
    pltpu.sync_copy(deg_sh.at[pl.ds(s * _RPT, _RPT)],
                    out_hbm.at[c, pl.ds(s * _RPT, _RPT)])


def _sc_degree(dstp, ones, zeros):
    mesh = plsc.VectorSubcoreMesh(core_axis_name="c", subcore_axis_name="s")
    f = pl.kernel(
        _degree_body,
        out_type=jax.ShapeDtypeStruct((_NC, _NP, _DW), jnp.float32),
        mesh=mesh,
        scratch_types=[
            pltpu.VMEM((_NB, _BB), jnp.int32),
            pltpu.VMEM((_BB, _DW), jnp.float32),
            pltpu.VMEM_SHARED((_NP, _DW), jnp.float32),
        ],
        compiler_params=pltpu.CompilerParams(use_tc_tiling_on_sc=False),
    )
    return f(dstp, ones, zeros)


_KB = 4
_NSUP = _NB // (2 * _KB)


def _scatter_body(y_hbm, src_hbm, dst_hbm, z_hbm, out_hbm,
                  src_v, dst_v, rows_a, rows_b, acc_sh, sga, sgb, ssa, ssb):
    c = lax.axis_index("c")
    s = lax.axis_index("s")
    wid = s * _NC + c
    pltpu.sync_copy(z_hbm.at[pl.ds(s * _RPT, _RPT)],
                    acc_sh.at[pl.ds(s * _RPT, _RPT)])
    pltpu.sync_copy(src_hbm.at[wid], src_v)
    pltpu.sync_copy(dst_hbm.at[wid], dst_v)
    plsc.subcore_barrier()

    def _gather(b, buf, sem, k):
        pltpu.async_copy(y_hbm.at[src_v.at[b]],
                         buf.at[pl.ds(k * _BB, _BB)], sem)

    def _gather_wait(buf, sem, k):
        pltpu.make_async_copy(y_hbm.at[src_v.at[0]],
                              buf.at[pl.ds(k * _BB, _BB)], sem).wait()

    def _scat(b, buf, sem, k):
        pltpu.async_copy(buf.at[pl.ds(k * _BB, _BB)],
                         acc_sh.at[dst_v.at[b]], sem, add=True)

    def _scat_wait(buf, sem, k):
        pltpu.make_async_copy(buf.at[pl.ds(k * _BB, _BB)],
                              acc_sh.at[dst_v.at[0]], sem).wait()

    for k in range(_KB):
        _gather(k, rows_a, sga, k)
        _gather(_KB + k, rows_b, sgb, k)

    def body(i, carry):
        b0 = i * (2 * _KB)
        for k in range(_KB):
            _gather_wait(rows_a, sga, k)
        for k in range(_KB):
            _scat(b0 + k, rows_a, ssa, k)
        for k in range(_KB):
            _gather_wait(rows_b, sgb, k)
        for k in range(_KB):
            _scat(b0 + _KB + k, rows_b, ssb, k)
        for k in range(_KB):
            _scat_wait(rows_a, ssa, k)

        @pl.when(i + 1 < _NSUP)
        def _():
            for k in range(_KB):
                _gather(b0 + 2 * _KB + k, rows_a, sga, k)

        for k in range(_KB):
            _scat_wait(rows_b, ssb, k)

        @pl.when(i + 1 < _NSUP)
        def _():
            for k in range(_KB):
                _gather(b0 + 3 * _KB + k, rows_b, sgb, k)

        return carry

    lax.fori_loop(0, _NSUP, body, 0)
    plsc.subcore_barrier()
    pltpu.sync_copy(acc_sh.at[pl.ds(s * _RPT, _RPT)],
                    out_hbm.at[c, pl.ds(s * _RPT, _RPT)])


def _sc_scatter(y, srcp, dstp, zeros):
    mesh = plsc.VectorSubcoreMesh(core_axis_name="c", subcore_axis_name="s")
    f = pl.kernel(
        _scatter_body,
        out_type=jax.ShapeDtypeStruct((_NC, _NP, _H), jnp.float32),
        mesh=mesh,
        scratch_types=[
            pltpu.VMEM((_NB, _BB), jnp.int32),
            pltpu.VMEM((_NB, _BB), jnp.int32),
            pltpu.VMEM((_KB * _BB, _H), jnp.float32),
            pltpu.VMEM((_KB * _BB, _H), jnp.float32),
            pltpu.VMEM_SHARED((_NP, _H), jnp.float32),
            pltpu.SemaphoreType.DMA,
            pltpu.SemaphoreType.DMA,
            pltpu.SemaphoreType.DMA,
            pltpu.SemaphoreType.DMA,
        ],
        compiler_params=pltpu.CompilerParams(use_tc_tiling_on_sc=False),
    )
    return f(y, srcp, dstp, zeros)


def _dot(a, b):
    return jnp.dot(a, b, preferred_element_type=jnp.float32)


def _fe_body(x_ref, deg_ref, w1_ref, b1_ref, w2_ref, b2_ref, w0_ref,
             dinv_ref, y0_ref):
    deg = deg_ref[0, :, 0:1] + deg_ref[1, :, 0:1] + 1.0
    dinv = lax.rsqrt(deg)
    h = jnp.maximum(_dot(x_ref[...], w1_ref[...]) + b1_ref[...], 0.0)
    h = _dot(h, w2_ref[...]) + b2_ref[...]
    dinv_ref[...] = dinv
    y0_ref[...] = dinv * _dot(h, w0_ref[...])


def _tc_fe(xp, degp, p):
    return pl.pallas_call(
        _fe_body,
        out_shape=[
            jax.ShapeDtypeStruct((_NP, 1), jnp.float32),
            jax.ShapeDtypeStruct((_NP, _H), jnp.float32),
        ],
    )(xp, degp, p['fe_W1'], p['fe_b1'].reshape(1, -1), p['fe_W2'],
      p['fe_b2'].reshape(1, -1), p['gcn_W0'])


def _bn_block(accp_ref, y_ref, dinv_ref, gb_ref, bng_ref, bnb_ref):
    dinv = dinv_ref[...]
    t = dinv * (accp_ref[0] + accp_ref[1] + y_ref[...]) + gb_ref[...]
    mask = (lax.broadcasted_iota(jnp.int32, (_NP, 1), 0) < _N
            ).astype(jnp.float32)
    m = jnp.sum(t * mask, axis=0, keepdims=True) * (1.0 / _N)
    d = (t - m) * mask
    v = jnp.sum(d * d, axis=0, keepdims=True) * (1.0 / _N)
    return bng_ref[...] * (t - m) * lax.rsqrt(v + 1e-5) + bnb_ref[...], dinv


def _mid_body(accp_ref, y_ref, dinv_ref, gb_ref, bng_ref, bnb_ref, wn_ref,
              ynext_ref):
    xh, dinv = _bn_block(accp_ref, y_ref, dinv_ref, gb_ref, bng_ref, bnb_ref)
    ynext_ref[...] = dinv * _dot(jnp.maximum(xh, 0.0), wn_ref[...])


def _tc_mid(accp, y, dinv, p, i):
    return pl.pallas_call(
        _mid_body,
        out_shape=jax.ShapeDtypeStruct((_NP, _H), jnp.float32),
    )(accp, y, dinv, p[f'gcn_b{i}'].reshape(1, -1),
      p[f'bn_g{i}'].reshape(1, -1), p[f'bn_b{i}'].reshape(1, -1),
      p[f'gcn_W{i + 1}'])


def _final_body(accp_ref, y_ref, dinv_ref, gb_ref, bng_ref, bnb_ref,
                cw1_ref, cb1_ref, cw2_ref, cb2_ref,
                fw1_ref, fb1_ref, fw2_ref, fb2_ref,
                rw1_ref, rb1_ref, rw2_ref, rb2_ref,
                logits_ref, conf_ref, corr_ref):
    hp, _ = _bn_block(accp_ref, y_ref, dinv_ref, gb_ref, bng_ref, bnb_ref)
    h = hp[:_N]
    cc = jnp.maximum(_dot(h, cw1_ref[...]) + cb1_ref[...], 0.0)
    logits_ref[...] = _dot(cc, cw2_ref[...]) + cb2_ref[...]
    cf = jnp.maximum(_dot(h, fw1_ref[...]) + fb1_ref[...], 0.0)
    conf_ref[...] = jax.nn.sigmoid(_dot(cf, fw2_ref[...]) + fb2_ref[...])
    cr = jnp.maximum(_dot(h, rw1_ref[...]) + rb1_ref[...], 0.0)
    corr_ref[...] = _dot(cr, rw2_ref[...]) + rb2_ref[...]


def _tc_final(accp, y, dinv, p):
    i = _L - 1
    return pl.pallas_call(
        _final_body,
        out_shape=[
            jax.ShapeDtypeStruct((_N, 3), jnp.float32),
            jax.ShapeDtypeStruct((_N, 1), jnp.float32),
            jax.ShapeDtypeStruct((_N, 1), jnp.float32),
        ],
    )(accp, y, dinv, p[f'gcn_b{i}'].reshape(1, -1),
      p[f'bn_g{i}'].reshape(1, -1), p[f'bn_b{i}'].reshape(1, -1),
      p['cls_W1'], p['cls_b1'].reshape(1, -1),
      p['cls_W2'], p['cls_b2'].reshape(1, -1),
      p['conf_W1'], p['conf_b1'].reshape(1, -1),
      p['conf_W2'], p['conf_b2'].reshape(1, -1),
      p['corr_W1'], p['corr_b1'].reshape(1, -1),
      p['corr_W2'], p['corr_b2'].reshape(1, -1))


def kernel(x, edge_index, params):
    p = params
    pad = _N + (jnp.arange(_EP - _E, dtype=jnp.int32) % (_NP - _N))
    srcp = jnp.concatenate([edge_index[0], pad]).reshape(_NW, _NB, _BB)
    dstp = jnp.concatenate([edge_index[1], pad]).reshape(_NW, _NB, _BB)
    xp = jnp.pad(x, ((0, _NP - _N), (0, 0)))
    zeros_deg = jnp.zeros((_NP, _DW), jnp.float32)
    ones_deg = jnp.ones((_BB, _DW), jnp.float32)
    zeros_acc = jnp.zeros((_NP, _H), jnp.float32)

    degp = _sc_degree(dstp, ones_deg, zeros_deg)
    dinv, y = _tc_fe(xp, degp, p)
    for i in range(_L - 1):
        accp = _sc_scatter(y, srcp, dstp, zeros_acc)
        y = _tc_mid(accp, y, dinv, p, i)
    accp = _sc_scatter(y, srcp, dstp, zeros_acc)
    logits, conf, corr = _tc_final(accp, y, dinv, p)
    return (logits, conf[:, 0], corr[:, 0])


# speedup vs baseline: 34.4506x; 1.2058x over previous
import functools

import jax
import jax.numpy as jnp
from jax import lax
from jax.experimental import pallas as pl
from jax.experimental.pallas import tpu as pltpu
import jax.experimental.pallas.tpu_sc as plsc

_N = 10000
_E = 320000
_IN = 128
_H = 64
_L = 4
_NP = 10240
_EP = 327680
_NC = 2
_NS = 16
_NW = _NC * _NS
_EPW = _EP // _NW
_BB = 128
_NB = _EPW // _BB
_RPT = _NP // _NS
_DW = 16


def _degree_body(dst_hbm, ones_hbm, z_hbm, out_hbm,
                 dst_v, ones_v, deg_buf, pair_buf, deg_sh):
    c = lax.axis_index("c")
    s = lax.axis_index("s")
    wid = s * _NC + c
    pltpu.sync_copy(z_hbm.at[pl.ds(s * _RPT, _RPT)],
                    deg_sh.at[pl.ds(s * _RPT, _RPT)])
    pltpu.sync_copy(dst_hbm.at[wid], dst_v)
    pltpu.sync_copy(ones_hbm, ones_v)
    plsc.subcore_barrier()

    def body(j, carry):
        pltpu.sync_copy(ones_v, deg_sh.at[dst_v.at[j]], add=True)
        return carry

    lax.fori_loop(0, _NB, body, 0)
    plsc.subcore_barrier()
    pltpu.sync_copy(deg_sh.at[pl.ds(s * _RPT, _RPT)], deg_buf)

    def expand(r, carry):
        v0 = deg_buf[2 * r]
        v1 = deg_buf[2 * r + 1]
        for q in range(4):
            pair_buf[r, pl.ds(q * 16, 16)] = v0
        for q in range(4):
            pair_buf[r, pl.ds(64 + q * 16, 16)] = v1
        return carry

    lax.fori_loop(0, _RPT // 2, expand, 0)
    pltpu.sync_copy(pair_buf, out_hbm.at[c, pl.ds(s * (_RPT // 2),
                                                  _RPT // 2)])


def _sc_degree(dstp, ones, zeros):
    mesh = plsc.VectorSubcoreMesh(core_axis_name="c", subcore_axis_name="s")
    f = pl.kernel(
        _degree_body,
        out_type=jax.ShapeDtypeStruct((_NC, _NP // 2, 2 * _H), jnp.float32),
        mesh=mesh,
        scratch_types=[
            pltpu.VMEM((_NB, _BB), jnp.int32),
            pltpu.VMEM((_BB, _DW), jnp.float32),
            pltpu.VMEM((_RPT, _DW), jnp.float32),
            pltpu.VMEM((_RPT // 2, 2 * _H), jnp.float32),
            pltpu.VMEM_SHARED((_NP, _DW), jnp.float32),
        ],
        compiler_params=pltpu.CompilerParams(use_tc_tiling_on_sc=False),
    )
    return f(dstp, ones, zeros)


_KB = 4
_NSUP = _NB // (2 * _KB)


def _scatter_body(y_hbm, src_hbm, dst_hbm, z_hbm, out_hbm,
                  src_v, dst_v, rows_a, rows_b, acc_sh, sga, sgb, ssa, ssb):
    c = lax.axis_index("c")
    s = lax.axis_index("s")
    wid = s * _NC + c
    pltpu.sync_copy(z_hbm.at[pl.ds(s * _RPT, _RPT)],
                    acc_sh.at[pl.ds(s * _RPT, _RPT)])
    pltpu.sync_copy(src_hbm.at[wid], src_v)
    pltpu.sync_copy(dst_hbm.at[wid], dst_v)
    plsc.subcore_barrier()

    def _gather(b, buf, sem, k):
        pltpu.async_copy(y_hbm.at[src_v.at[b]],
                         buf.at[pl.ds(k * _BB, _BB)], sem)

    def _gather_wait(buf, sem, k):
        pltpu.make_async_copy(y_hbm.at[src_v.at[0]],
                              buf.at[pl.ds(k * _BB, _BB)], sem).wait()

    def _scat(b, buf, sem, k):
        pltpu.async_copy(buf.at[pl.ds(k * _BB, _BB)],
                         acc_sh.at[dst_v.at[b]], sem, add=True)

    def _scat_wait(buf, sem, k):
        pltpu.make_async_copy(buf.at[pl.ds(k * _BB, _BB)],
                              acc_sh.at[dst_v.at[0]], sem).wait()

    for k in range(_KB):
        _gather(k, rows_a, sga, k)
        _gather(_KB + k, rows_b, sgb, k)

    def body(i, carry):
        b0 = i * (2 * _KB)
        for k in range(_KB):
            _gather_wait(rows_a, sga, k)
        for k in range(_KB):
            _scat(b0 + k, rows_a, ssa, k)
        for k in range(_KB):
            _gather_wait(rows_b, sgb, k)
        for k in range(_KB):
            _scat(b0 + _KB + k, rows_b, ssb, k)
        for k in range(_KB):
            _scat_wait(rows_a, ssa, k)

        @pl.when(i + 1 < _NSUP)
        def _():
            for k in range(_KB):
                _gather(b0 + 2 * _KB + k, rows_a, sga, k)

        for k in range(_KB):
            _scat_wait(rows_b, ssb, k)

        @pl.when(i + 1 < _NSUP)
        def _():
            for k in range(_KB):
                _gather(b0 + 3 * _KB + k, rows_b, sgb, k)

        return carry

    lax.fori_loop(0, _NSUP, body, 0)
    plsc.subcore_barrier()
    pltpu.sync_copy(acc_sh.at[pl.ds(s * _RPT, _RPT)],
                    out_hbm.at[c, pl.ds(s * _RPT, _RPT)])


def _sc_scatter(y, srcp, dstp, zeros):
    mesh = plsc.VectorSubcoreMesh(core_axis_name="c", subcore_axis_name="s")
    f = pl.kernel(
        _scatter_body,
        out_type=jax.ShapeDtypeStruct((_NC, _NP, _H), jnp.float32),
        mesh=mesh,
        scratch_types=[
            pltpu.VMEM((_NB, _BB), jnp.int32),
            pltpu.VMEM((_NB, _BB), jnp.int32),
            pltpu.VMEM((_KB * _BB, _H), jnp.float32),
            pltpu.VMEM((_KB * _BB, _H), jnp.float32),
            pltpu.VMEM_SHARED((_NP, _H), jnp.float32),
            pltpu.SemaphoreType.DMA,
            pltpu.SemaphoreType.DMA,
            pltpu.SemaphoreType.DMA,
            pltpu.SemaphoreType.DMA,
        ],
        compiler_params=pltpu.CompilerParams(use_tc_tiling_on_sc=False),
    )
    return f(y, srcp, dstp, zeros)


_NH = _NP // 2
_NR = _N // 2


def _dot(a, b):
    return jnp.dot(a, b, preferred_element_type=jnp.float32)


def _fe_body(x_ref, deg_ref, w1_ref, b1_ref, w2_ref, b2_ref, w0_ref,
             dinv_ref, y0_ref):
    dinv = lax.rsqrt(deg_ref[0] + deg_ref[1] + 1.0)
    h = jnp.maximum(_dot(x_ref[...], w1_ref[...]) + b1_ref[...], 0.0)
    h = _dot(h, w2_ref[...]) + b2_ref[...]
    dinv_ref[...] = dinv
    y0_ref[...] = dinv * _dot(h, w0_ref[...])


def _tc_fe(x2, degp, bd):
    return pl.pallas_call(
        _fe_body,
        out_shape=[
            jax.ShapeDtypeStruct((_NH, 2 * _H), jnp.float32),
            jax.ShapeDtypeStruct((_NH, 2 * _H), jnp.float32),
        ],
    )(x2, degp, bd['fe_W1'], bd['fe_b1'], bd['fe_W2'], bd['fe_b2'],
      bd['gcn_W0'])


def _bn_block(accp_ref, y_ref, dinv_ref, gb_ref, bng_ref, bnb_ref, fold_ref):
    dinv = dinv_ref[...]
    t = dinv * (accp_ref[0] + accp_ref[1] + y_ref[...]) + gb_ref[...]
    mask = (lax.broadcasted_iota(jnp.int32, (_NH, 1), 0) < _NR
            ).astype(jnp.float32)
    fold = fold_ref[...]
    m = jnp.dot(jnp.sum(t * mask, axis=0, keepdims=True) * (1.0 / _N),
                fold, precision=lax.Precision.HIGHEST,
                preferred_element_type=jnp.float32)
    d = (t - m) * mask
    v = jnp.dot(jnp.sum(d * d, axis=0, keepdims=True) * (1.0 / _N),
                fold, precision=lax.Precision.HIGHEST,
                preferred_element_type=jnp.float32)
    return bng_ref[...] * (t - m) * lax.rsqrt(v + 1e-5) + bnb_ref[...], dinv


def _mid_body(accp_ref, y_ref, dinv_ref, fold_ref, gb_ref, bng_ref, bnb_ref,
              wn_ref, ynext_ref):
    xh, dinv = _bn_block(accp_ref, y_ref, dinv_ref, gb_ref, bng_ref,
                         bnb_ref, fold_ref)
    ynext_ref[...] = dinv * _dot(jnp.maximum(xh, 0.0), wn_ref[...])


def _tc_mid(accp, y, dinv, fold, bd, i):
    return pl.pallas_call(
        _mid_body,
        out_shape=jax.ShapeDtypeStruct((_NH, 2 * _H), jnp.float32),
    )(accp, y, dinv, fold, bd[f'gcn_b{i}'], bd[f'bn_g{i}'], bd[f'bn_b{i}'],
      bd[f'gcn_W{i + 1}'])


def _final_body(accp_ref, y_ref, dinv_ref, fold_ref, gb_ref, bng_ref,
                bnb_ref, cw1_ref, cb1_ref, cw2_ref, cb2_ref,
                fw1_ref, fb1_ref, fw2_ref, fb2_ref,
                rw1_ref, rb1_ref, rw2_ref, rb2_ref,
                logits_ref, conf_ref, corr_ref):
    h, _ = _bn_block(accp_ref, y_ref, dinv_ref, gb_ref, bng_ref, bnb_ref,
                     fold_ref)
    cc = jnp.maximum(_dot(h, cw1_ref[...]) + cb1_ref[...], 0.0)
    logits_ref[...] = _dot(cc, cw2_ref[...]) + cb2_ref[...]
    cf = jnp.maximum(_dot(h, fw1_ref[...]) + fb1_ref[...], 0.0)
    conf_ref[...] = jax.nn.sigmoid(_dot(cf, fw2_ref[...]) + fb2_ref[...])
    cr = jnp.maximum(_dot(h, rw1_ref[...]) + rb1_ref[...], 0.0)
    corr_ref[...] = _dot(cr, rw2_ref[...]) + rb2_ref[...]


def _tc_final(accp, y, dinv, fold, bd, i):
    return pl.pallas_call(
        _final_body,
        out_shape=[
            jax.ShapeDtypeStruct((_NH, 6), jnp.float32),
            jax.ShapeDtypeStruct((_NH, 2), jnp.float32),
            jax.ShapeDtypeStruct((_NH, 2), jnp.float32),
        ],
    )(accp, y, dinv, fold, bd[f'gcn_b{i}'], bd[f'bn_g{i}'], bd[f'bn_b{i}'],
      bd['cls_W1'], bd['cls_b1'], bd['cls_W2'], bd['cls_b2'],
      bd['conf_W1'], bd['conf_b1'], bd['conf_W2'], bd['conf_b2'],
      bd['corr_W1'], bd['corr_b1'], bd['corr_W2'], bd['corr_b2'])


def _bdiag(w):
    z1 = jnp.zeros_like(w)
    return jnp.concatenate(
        [jnp.concatenate([w, z1], axis=1),
         jnp.concatenate([z1, w], axis=1)], axis=0)


def _pairb(b):
    return jnp.concatenate([b, b]).reshape(1, -1)


def kernel(x, edge_index, params):
    p = params
    pad = _N + (jnp.arange(_EP - _E, dtype=jnp.int32) % (_NP - _N))
    srcp = jnp.concatenate([edge_index[0], pad]).reshape(_NW, _NB, _BB)
    dstp = jnp.concatenate([edge_index[1], pad]).reshape(_NW, _NB, _BB)
    x2 = jnp.pad(x, ((0, _NP - _N), (0, 0))).reshape(_NH, 2 * _IN)
    zeros_deg = jnp.zeros((_NP, _DW), jnp.float32)
    ones_deg = jnp.ones((_BB, _DW), jnp.float32)
    zeros_acc = jnp.zeros((_NP, _H), jnp.float32)
    fold = jnp.tile(jnp.eye(_H, dtype=jnp.float32), (2, 2))
    bd = {}
    for k, w in p.items():
        bd[k] = _bdiag(w) if w.ndim == 2 else _pairb(w)

    degp = _sc_degree(dstp, ones_deg, zeros_deg)
    dinv, y = _tc_fe(x2, degp, bd)
    for i in range(_L - 1):
        accp = _sc_scatter(y.reshape(_NP, _H), srcp, dstp, zeros_acc)
        y = _tc_mid(accp.reshape(_NC, _NH, 2 * _H), y, dinv, fold, bd, i)
    accp = _sc_scatter(y.reshape(_NP, _H), srcp, dstp, zeros_acc)
    logits, conf, corr = _tc_final(accp.reshape(_NC, _NH, 2 * _H), y, dinv,
                                   fold, bd, _L - 1)
    return (logits.reshape(_NP, 3)[:_N],
            conf.reshape(_NP)[:_N],
            corr.reshape(_NP)[:_N])
